# baseline (speedup 1.0000x reference)
import functools

import jax
import jax.numpy as jnp
from jax import lax
from jax.experimental import pallas as pl
from jax.experimental.pallas import tpu as pltpu
from jax.experimental.pallas import tpu_sc as plsc

_VOCAB = 1000000
_DIM = 64
_PITCH = 128
_BATCH = 16384
_NEG = 5

_NC = 2
_NS = 16
_NW = _NC * _NS
_BPW = _BATCH // _NW
_CHUNK = 64
_NCHUNK = _BPW // _CHUNK
_GROUPS = _CHUNK // 16

_TBLK = 10240
_TSUB = 2048


_NSUPER = (_VOCAB + 2 * _TBLK - 1) // (2 * _TBLK)
_NQ = _NSUPER * _TBLK


def _tc_repack(Wa_t, Wb_t):
  def _pack(a, b):
    wa = jax.lax.bitcast_convert_type(
        a.astype(jnp.bfloat16), jnp.uint16).astype(jnp.uint32)
    wb = jax.lax.bitcast_convert_type(
        b.astype(jnp.bfloat16), jnp.uint16).astype(jnp.uint32)
    return jax.lax.bitcast_convert_type(wa | (wb << 16), jnp.int32)

  def body(a1_ref, b1_ref, a2_ref, b2_ref, o_ref):
    for j in range(_TBLK // _TSUB):
      sl = pl.ds(j * _TSUB, _TSUB)
      w1 = _pack(a1_ref[:, sl], b1_ref[:, sl]).T
      w2 = _pack(a2_ref[:, sl], b2_ref[:, sl]).T
      o_ref[sl, :] = jnp.concatenate([w1, w2], axis=1)

  last_blk = (_VOCAB + _TBLK - 1) // _TBLK - 1
  lo_spec = pl.BlockSpec((_DIM, _TBLK), lambda i: (0, 2 * i))
  hi_spec = pl.BlockSpec((_DIM, _TBLK),
                         lambda i: (0, jnp.minimum(2 * i + 1, last_blk)))
  out_spec = pl.BlockSpec((_TBLK, _PITCH), lambda i: (i, 0))
  return pl.pallas_call(
      body,
      grid=(_NSUPER,),
      in_specs=[lo_spec, lo_spec, hi_spec, hi_spec],
      out_specs=out_spec,
      out_shape=jax.ShapeDtypeStruct((_NQ, _PITCH), jnp.int32),
  )(Wa_t, Wb_t, Wa_t, Wb_t)


def _sc_products(qin, qcw, qneg, pin, pcw, pneg, Wp):
  mesh = plsc.VectorSubcoreMesh(core_axis_name="c", subcore_axis_name="s")

  @functools.partial(
      pl.kernel,
      out_type=jax.ShapeDtypeStruct((1 + _NEG, _BATCH), jnp.float32),
      mesh=mesh,
      scratch_types=[
          pltpu.VMEM((_BPW,), jnp.int32),
          pltpu.VMEM((_BPW,), jnp.int32),
          pltpu.VMEM((_BPW * _NEG,), jnp.int32),
          pltpu.VMEM((_BPW,), jnp.int32),
          pltpu.VMEM((_BPW,), jnp.int32),
          pltpu.VMEM((_BPW * _NEG,), jnp.int32),
          pltpu.VMEM((_CHUNK, _PITCH), jnp.int32),
          pltpu.VMEM((_CHUNK, _PITCH), jnp.int32),
          pltpu.VMEM((_CHUNK * _NEG, _PITCH), jnp.int32),
          pltpu.VMEM((_CHUNK, _PITCH), jnp.int32),
          pltpu.VMEM((_CHUNK, _PITCH), jnp.int32),
          pltpu.VMEM((_CHUNK * _NEG, _PITCH), jnp.int32),
          pltpu.VMEM((1 + _NEG, _BPW), jnp.float32),
          pltpu.SemaphoreType.DMA,
          pltpu.SemaphoreType.DMA,
      ],
      compiler_params=pltpu.CompilerParams(needs_layout_passes=False),
  )
  def k(qin_hbm, qcw_hbm, qng_hbm, pin_hbm, pcw_hbm, png_hbm, wp_hbm, out_hbm,
        q_in, q_ctx, q_neg, p_in, p_ctx, p_neg,
        ri0, rc0, rn0, ri1, rc1, rn1, prod, semA, semB):
    wid = lax.axis_index("s") * _NC + lax.axis_index("c")
    base = wid * _BPW
    pltpu.sync_copy(qin_hbm.at[pl.ds(base, _BPW)], q_in)
    pltpu.sync_copy(qcw_hbm.at[pl.ds(base, _BPW)], q_ctx)
    pltpu.sync_copy(qng_hbm.at[pl.ds(base * _NEG, _BPW * _NEG)], q_neg)
    pltpu.sync_copy(pin_hbm.at[pl.ds(base, _BPW)], p_in)
    pltpu.sync_copy(pcw_hbm.at[pl.ds(base, _BPW)], p_ctx)
    pltpu.sync_copy(png_hbm.at[pl.ds(base * _NEG, _BPW * _NEG)], p_neg)
    lanes = lax.iota(jnp.int32, 16)

    def fire(c, ri, rc, rn, sem):
      pltpu.async_copy(wp_hbm.at[q_in.at[pl.ds(c * _CHUNK, _CHUNK)]], ri, sem)
      pltpu.async_copy(wp_hbm.at[q_ctx.at[pl.ds(c * _CHUNK, _CHUNK)]], rc, sem)
      pltpu.async_copy(
          wp_hbm.at[q_neg.at[pl.ds(c * _CHUNK * _NEG, _CHUNK * _NEG)]], rn, sem)

    def drain(ri, rc, rn, sem):
      pltpu.make_async_copy(wp_hbm.at[pl.ds(0, _CHUNK)], ri, sem).wait()
      pltpu.make_async_copy(wp_hbm.at[pl.ds(0, _CHUNK)], rc, sem).wait()
      pltpu.make_async_copy(wp_hbm.at[pl.ds(0, _CHUNK * _NEG)], rn, sem).wait()

    def compute(c, ri, rc, rn):
      def group_body(g, carry):
        e0 = c * _CHUNK + g * 16
        rows_e = g * 16 + lanes
        rows_n = (g * 16 + lanes) * _NEG
        pa = plsc.load_gather(p_in, [e0 + lanes])
        pc = plsc.load_gather(p_ctx, [e0 + lanes])
        pn = [plsc.load_gather(p_neg, [(e0 + lanes) * _NEG + kk])
              for kk in range(_NEG)]
        acc = [jnp.zeros((16,), jnp.float32) for _ in range(1 + _NEG)]
        for t in range(_DIM):
          off = (lanes + t) & (_DIM - 1)
          wa = plsc.load_gather(ri, [rows_e, pa + off])
          wc = plsc.load_gather(rc, [rows_e, pc + off])
          a = plsc.bitcast(wa << 16, jnp.float32)
          cv = plsc.bitcast(wc & jnp.int32(-65536), jnp.float32)
          acc[0] = acc[0] + a * cv
          for kk in range(_NEG):
            wn = plsc.load_gather(rn, [rows_n + kk, pn[kk] + off])
            acc[1 + kk] = acc[1 + kk] + plsc.bitcast(wn << 16, jnp.float32) * a
        prod[0, pl.ds(e0, 16)] = acc[0]
        for kk in range(_NEG):
          prod[1 + kk, pl.ds(e0, 16)] = -acc[1 + kk]
        return carry

      lax.fori_loop(0, _GROUPS, group_body, 0)

    fire(0, ri0, rc0, rn0, semA)

    def pair_body(h, carry0):
      c0 = 2 * h
      fire(c0 + 1, ri1, rc1, rn1, semB)
      drain(ri0, rc0, rn0, semA)
      compute(c0, ri0, rc0, rn0)

      @pl.when(c0 + 2 < _NCHUNK)
      def _():
        fire(c0 + 2, ri0, rc0, rn0, semA)

      drain(ri1, rc1, rn1, semB)
      compute(c0 + 1, ri1, rc1, rn1)
      return carry0

    lax.fori_loop(0, _NCHUNK // 2, pair_body, 0)
    pltpu.sync_copy(prod, out_hbm.at[:, pl.ds(base, _BPW)])

  return k(qin, qcw, qneg, pin, pcw, pneg, Wp)


def _tc_loss(prods):
  def body(p_ref, o_ref):
    x = p_ref[...]
    ls = jnp.minimum(x, 0.0) - jnp.log1p(jnp.exp(-jnp.abs(x)))
    o_ref[0, 0] = -jnp.sum(ls) / _BATCH

  return pl.pallas_call(
      body,
      out_shape=jax.ShapeDtypeStruct((1, 1), jnp.float32),
      out_specs=pl.BlockSpec(memory_space=pltpu.SMEM),
  )(prods)


def kernel(input_word, context_word, W_in, W_ctx):
  neg_idx = jax.random.randint(jax.random.key(1), (_BATCH, _NEG), 0, _VOCAB)
  neg_flat = neg_idx.reshape(-1).astype(jnp.int32)
  iw = input_word.astype(jnp.int32)
  cw = context_word.astype(jnp.int32)
  Wp = _tc_repack(W_in.T, W_ctx.T)

  def to_q(r):
    return (r // (2 * _TBLK)) * _TBLK + (r % _TBLK)

  def to_p(r):
    return ((r // _TBLK) & 1) * _DIM

  prods = _sc_products(to_q(iw), to_q(cw), to_q(neg_flat),
                       to_p(iw), to_p(cw), to_p(neg_flat), Wp)
  return _tc_loss(prods)[0, 0]


# speedup vs baseline: 1.1379x; 1.1379x over previous
import functools

import jax
import jax.numpy as jnp
from jax import lax
from jax.experimental import pallas as pl
from jax.experimental.pallas import tpu as pltpu
from jax.experimental.pallas import tpu_sc as plsc

_VOCAB = 1000000
_DIM = 64
_PITCH = 128
_BATCH = 16384
_NEG = 5

_NC = 2
_NS = 16
_NW = _NC * _NS
_BPW = _BATCH // _NW
_CHUNK = 64
_NCHUNK = _BPW // _CHUNK
_GROUPS = _CHUNK // 16

_TBLK = 8192
_TSUB = 2048


_NSUPER = (_VOCAB + 2 * _TBLK - 1) // (2 * _TBLK)
_NQ = _NSUPER * _TBLK


def _tc_repack(Wa_t, Wb_t):
  def _pack(a, b):
    wa = jax.lax.bitcast_convert_type(
        a.astype(jnp.bfloat16), jnp.uint16).astype(jnp.uint32)
    wb = jax.lax.bitcast_convert_type(
        b.astype(jnp.bfloat16), jnp.uint16).astype(jnp.uint32)
    return jax.lax.bitcast_convert_type(wa | (wb << 16), jnp.int32)

  def body(a1_ref, b1_ref, a2_ref, b2_ref, o_ref):
    for j in range(_TBLK // _TSUB):
      sl = pl.ds(j * _TSUB, _TSUB)
      w1 = _pack(a1_ref[:, sl], b1_ref[:, sl]).T
      w2 = _pack(a2_ref[:, sl], b2_ref[:, sl]).T
      o_ref[sl, :] = jnp.concatenate([w1, w2], axis=1)

  last_blk = (_VOCAB + _TBLK - 1) // _TBLK - 1
  lo_spec = pl.BlockSpec((_DIM, _TBLK), lambda i: (0, 2 * i))
  hi_spec = pl.BlockSpec((_DIM, _TBLK),
                         lambda i: (0, jnp.minimum(2 * i + 1, last_blk)))
  out_spec = pl.BlockSpec((_TBLK, _PITCH), lambda i: (i, 0))
  return pl.pallas_call(
      body,
      grid=(_NSUPER,),
      in_specs=[lo_spec, lo_spec, hi_spec, hi_spec],
      out_specs=out_spec,
      out_shape=jax.ShapeDtypeStruct((_NQ, _PITCH), jnp.int32),
  )(Wa_t, Wb_t, Wa_t, Wb_t)


def _sc_products(qin, qcw, qneg, pin, pcw, pneg, Wp):
  mesh = plsc.VectorSubcoreMesh(core_axis_name="c", subcore_axis_name="s")

  @functools.partial(
      pl.kernel,
      out_type=jax.ShapeDtypeStruct((1 + _NEG, _BATCH), jnp.float32),
      mesh=mesh,
      scratch_types=[
          pltpu.VMEM((_BPW,), jnp.int32),
          pltpu.VMEM((_BPW,), jnp.int32),
          pltpu.VMEM((_BPW * _NEG,), jnp.int32),
          pltpu.VMEM((_BPW,), jnp.int32),
          pltpu.VMEM((_BPW,), jnp.int32),
          pltpu.VMEM((_BPW * _NEG,), jnp.int32),
          pltpu.VMEM((_CHUNK, _PITCH), jnp.int32),
          pltpu.VMEM((_CHUNK, _PITCH), jnp.int32),
          pltpu.VMEM((_CHUNK * _NEG, _PITCH), jnp.int32),
          pltpu.VMEM((_CHUNK, _PITCH), jnp.int32),
          pltpu.VMEM((_CHUNK, _PITCH), jnp.int32),
          pltpu.VMEM((_CHUNK * _NEG, _PITCH), jnp.int32),
          pltpu.VMEM((1 + _NEG, _BPW), jnp.float32),
          pltpu.SemaphoreType.DMA,
          pltpu.SemaphoreType.DMA,
      ],
      compiler_params=pltpu.CompilerParams(needs_layout_passes=False),
  )
  def k(qin_hbm, qcw_hbm, qng_hbm, pin_hbm, pcw_hbm, png_hbm, wp_hbm, out_hbm,
        q_in, q_ctx, q_neg, p_in, p_ctx, p_neg,
        ri0, rc0, rn0, ri1, rc1, rn1, prod, semA, semB):
    wid = lax.axis_index("s") * _NC + lax.axis_index("c")
    base = wid * _BPW
    pltpu.sync_copy(qin_hbm.at[pl.ds(base, _BPW)], q_in)
    pltpu.sync_copy(qcw_hbm.at[pl.ds(base, _BPW)], q_ctx)
    pltpu.sync_copy(qng_hbm.at[pl.ds(base * _NEG, _BPW * _NEG)], q_neg)
    pltpu.sync_copy(pin_hbm.at[pl.ds(base, _BPW)], p_in)
    pltpu.sync_copy(pcw_hbm.at[pl.ds(base, _BPW)], p_ctx)
    pltpu.sync_copy(png_hbm.at[pl.ds(base * _NEG, _BPW * _NEG)], p_neg)
    lanes = lax.iota(jnp.int32, 16)

    def fire(c, ri, rc, rn, sem):
      pltpu.async_copy(wp_hbm.at[q_in.at[pl.ds(c * _CHUNK, _CHUNK)]], ri, sem)
      pltpu.async_copy(wp_hbm.at[q_ctx.at[pl.ds(c * _CHUNK, _CHUNK)]], rc, sem)
      pltpu.async_copy(
          wp_hbm.at[q_neg.at[pl.ds(c * _CHUNK * _NEG, _CHUNK * _NEG)]], rn, sem)

    def drain(ri, rc, rn, sem):
      pltpu.make_async_copy(wp_hbm.at[pl.ds(0, _CHUNK)], ri, sem).wait()
      pltpu.make_async_copy(wp_hbm.at[pl.ds(0, _CHUNK)], rc, sem).wait()
      pltpu.make_async_copy(wp_hbm.at[pl.ds(0, _CHUNK * _NEG)], rn, sem).wait()

    def compute(c, ri, rc, rn):
      def group_body(g, carry):
        e0 = c * _CHUNK + g * 16
        rows_e = g * 16 + lanes
        rows_n = (g * 16 + lanes) * _NEG
        pa = plsc.load_gather(p_in, [e0 + lanes])
        pc = plsc.load_gather(p_ctx, [e0 + lanes])
        pn = [plsc.load_gather(p_neg, [(e0 + lanes) * _NEG + kk])
              for kk in range(_NEG)]
        acc = [jnp.zeros((16,), jnp.float32) for _ in range(1 + _NEG)]
        for t in range(_DIM):
          off = (lanes + t) & (_DIM - 1)
          wa = plsc.load_gather(ri, [rows_e, pa + off])
          wc = plsc.load_gather(rc, [rows_e, pc + off])
          a = plsc.bitcast(wa << 16, jnp.float32)
          cv = plsc.bitcast(wc & jnp.int32(-65536), jnp.float32)
          acc[0] = acc[0] + a * cv
          for kk in range(_NEG):
            wn = plsc.load_gather(rn, [rows_n + kk, pn[kk] + off])
            acc[1 + kk] = acc[1 + kk] + plsc.bitcast(wn << 16, jnp.float32) * a
        prod[0, pl.ds(e0, 16)] = acc[0]
        for kk in range(_NEG):
          prod[1 + kk, pl.ds(e0, 16)] = -acc[1 + kk]
        return carry

      lax.fori_loop(0, _GROUPS, group_body, 0)

    fire(0, ri0, rc0, rn0, semA)

    def pair_body(h, carry0):
      c0 = 2 * h
      fire(c0 + 1, ri1, rc1, rn1, semB)
      drain(ri0, rc0, rn0, semA)
      compute(c0, ri0, rc0, rn0)

      @pl.when(c0 + 2 < _NCHUNK)
      def _():
        fire(c0 + 2, ri0, rc0, rn0, semA)

      drain(ri1, rc1, rn1, semB)
      compute(c0 + 1, ri1, rc1, rn1)
      return carry0

    lax.fori_loop(0, _NCHUNK // 2, pair_body, 0)
    pltpu.sync_copy(prod, out_hbm.at[:, pl.ds(base, _BPW)])

  return k(qin, qcw, qneg, pin, pcw, pneg, Wp)


def _tc_loss(prods):
  def body(p_ref, o_ref):
    x = p_ref[...]
    ls = jnp.minimum(x, 0.0) - jnp.log1p(jnp.exp(-jnp.abs(x)))
    o_ref[0, 0] = -jnp.sum(ls) / _BATCH

  return pl.pallas_call(
      body,
      out_shape=jax.ShapeDtypeStruct((1, 1), jnp.float32),
      out_specs=pl.BlockSpec(memory_space=pltpu.SMEM),
  )(prods)


def kernel(input_word, context_word, W_in, W_ctx):
  neg_idx = jax.random.randint(jax.random.key(1), (_BATCH, _NEG), 0, _VOCAB)
  neg_flat = neg_idx.reshape(-1).astype(jnp.int32)
  iw = input_word.astype(jnp.int32)
  cw = context_word.astype(jnp.int32)
  Wp = _tc_repack(W_in.T, W_ctx.T)

  def to_q(r):
    return (r // (2 * _TBLK)) * _TBLK + (r % _TBLK)

  def to_p(r):
    return ((r // _TBLK) & 1) * _DIM

  prods = _sc_products(to_q(iw), to_q(cw), to_q(neg_flat),
                       to_p(iw), to_p(cw), to_p(neg_flat), Wp)
  return _tc_loss(prods)[0, 0]
